# baseline (speedup 1.0000x reference)
import functools

import jax
import jax.numpy as jnp
from jax import lax
from jax.experimental import pallas as pl
from jax.experimental.pallas import tpu as pltpu
from jax.experimental.pallas import tpu_sc as plsc

N = 20000
G = 128
L = 16
GJ = G // L
NC = 2
NS = 16
NW = NC * NS
CHUNK = 640
NPAD = NW * CHUNK

_mesh = plsc.VectorSubcoreMesh(
    core_axis_name="c", subcore_axis_name="s", num_cores=NC, num_subcores=NS)


TAIL = N - (NW - 1) * CHUNK


def _worker_base():
    wid = lax.axis_index("s") * NC + lax.axis_index("c")
    return wid, wid * CHUNK


def _chunk_out(v_ref, hbm_ref, base, wid):
    @pl.when(wid < NW - 1)
    def _():
        pltpu.sync_copy(v_ref, hbm_ref.at[pl.ds(base, CHUNK)])

    @pl.when(wid == NW - 1)
    def _():
        pltpu.sync_copy(v_ref.at[pl.ds(0, TAIL)], hbm_ref.at[pl.ds(base, TAIL)])


def _chunk_in(hbm_ref, v_ref, base, wid):
    @pl.when(wid < NW - 1)
    def _():
        pltpu.sync_copy(hbm_ref.at[pl.ds(base, CHUNK)], v_ref)

    @pl.when(wid == NW - 1)
    def _():
        pltpu.sync_copy(hbm_ref.at[pl.ds(base, TAIL)], v_ref.at[pl.ds(0, TAIL)])


def _ids_plus1(j):
    return lax.iota(jnp.int32, L) * GJ + (j + 1)


def _lane_max(x):
    iota = lax.iota(jnp.int32, L)
    for k in (8, 4, 2, 1):
        perm = jnp.bitwise_xor(iota, k)
        x = jnp.maximum(
            x, jnp.take_along_axis(x, perm, axis=0, mode="promise_in_bounds"))
    return x


def _phase1_body(boxes_hbm, gt_hbm, mo_hbm, asg_hbm, cm_hbm, ca_hbm,
                 bx_v, gt_v, ga_v, rm_v, as_v, cm_v, ca_v, sem):
    wid, base = _worker_base()
    c1 = pltpu.async_copy(boxes_hbm.at[:, pl.ds(base, CHUNK)], bx_v, sem)
    c2 = pltpu.async_copy(gt_hbm, gt_v, sem)
    c1.wait()
    c2.wait()

    iota = lax.iota(jnp.int32, L)
    for j in range(GJ):
        s = pl.ds(j * L, L)
        ga_v[s] = (gt_v[2, s] - gt_v[0, s]) * (gt_v[3, s] - gt_v[1, s])
        ca_v[s] = jnp.zeros((L,), jnp.int32)

    init_cm = tuple(jnp.full((L,), -1.0, jnp.float32) for _ in range(GJ))

    def do_group(tb, cms):
        bx1g = bx_v[0, pl.ds(tb, L)]
        by1g = bx_v[1, pl.ds(tb, L)]
        bx2g = bx_v[2, pl.ds(tb, L)]
        by2g = bx_v[3, pl.ds(tb, L)]
        rm_acc = jnp.zeros((L,), jnp.float32)
        as_acc = jnp.zeros((L,), jnp.int32)
        cms = list(cms)
        for half in range(L // 2):
            la, lb = 2 * half, 2 * half + 1
            ax1 = jnp.full((L,), bx1g[la])
            ay1 = jnp.full((L,), by1g[la])
            ax2 = jnp.full((L,), bx2g[la])
            ay2 = jnp.full((L,), by2g[la])
            bx1 = jnp.full((L,), bx1g[lb])
            by1 = jnp.full((L,), by1g[lb])
            bx2 = jnp.full((L,), bx2g[lb])
            by2 = jnp.full((L,), by2g[lb])
            area_a = (ax2 - ax1) * (ay2 - ay1)
            area_b = (bx2 - bx1) * (by2 - by1)
            bidxa = jnp.full((L,), base + tb + la, dtype=jnp.int32)
            bidxb = jnp.full((L,), base + tb + lb, dtype=jnp.int32)
            best_a = best_b = None
            barg_a = barg_b = None
            for j in range(GJ):
                s = pl.ds(j * L, L)
                gx1 = gt_v[0, s]
                gy1 = gt_v[1, s]
                gx2 = gt_v[2, s]
                gy2 = gt_v[3, s]
                ga = ga_v[s]
                ids1 = _ids_plus1(j)

                wa = jnp.maximum(jnp.minimum(ax2, gx2) - jnp.maximum(ax1, gx1),
                                 0.0)
                ha = jnp.maximum(jnp.minimum(ay2, gy2) - jnp.maximum(ay1, gy1),
                                 0.0)
                inter_a = wa * ha
                iou_a = inter_a / (area_a + ga - inter_a)
                ma = iou_a > cms[j]
                cms[j] = jnp.where(ma, iou_a, cms[j])
                if j == 0:
                    best_a, barg_a = iou_a, ids1
                else:
                    ra = iou_a > best_a
                    best_a = jnp.where(ra, iou_a, best_a)
                    barg_a = jnp.where(ra, ids1, barg_a)

                wb = jnp.maximum(jnp.minimum(bx2, gx2) - jnp.maximum(bx1, gx1),
                                 0.0)
                hb = jnp.maximum(jnp.minimum(by2, gy2) - jnp.maximum(by1, gy1),
                                 0.0)
                inter_b = wb * hb
                iou_b = inter_b / (area_b + ga - inter_b)
                mb = iou_b > cms[j]
                cms[j] = jnp.where(mb, iou_b, cms[j])
                ca_v[s] = jnp.where(
                    mb, bidxb, jnp.where(ma, bidxa, ca_v[s]))
                if j == 0:
                    best_b, barg_b = iou_b, ids1
                else:
                    rb = iou_b > best_b
                    best_b = jnp.where(rb, iou_b, best_b)
                    barg_b = jnp.where(rb, ids1, barg_b)

            for lane, best, barg in ((la, best_a, barg_a),
                                     (lb, best_b, barg_b)):
                rmx = _lane_max(best)
                ffs = plsc.all_reduce_ffs(best == rmx)
                argsel = jnp.take_along_axis(barg, ffs, axis=0,
                                             mode="promise_in_bounds")
                asv = jnp.where(rmx >= 0.5, argsel, 0)
                sel = iota == lane
                rm_acc = jnp.where(sel, rmx, rm_acc)
                as_acc = jnp.where(sel, asv, as_acc)
        rm_v[pl.ds(tb, L)] = rm_acc
        as_v[pl.ds(tb, L)] = as_acc
        return tuple(cms)

    def group(t, cms):
        cms = do_group(t * (2 * L), cms)
        return do_group(t * (2 * L) + L, cms)

    cms = lax.fori_loop(0, CHUNK // L // 2, group, init_cm)
    for j in range(GJ):
        cm_v[pl.ds(j * L, L)] = cms[j]
    o3 = pltpu.async_copy(cm_v, cm_hbm.at[wid], sem)
    o4 = pltpu.async_copy(ca_v, ca_hbm.at[wid], sem)
    _chunk_out(rm_v, mo_hbm, base, wid)
    _chunk_out(as_v, asg_hbm, base, wid)
    o3.wait()
    o4.wait()


_phase1 = functools.partial(
    pl.kernel,
    out_type=(
        jax.ShapeDtypeStruct((N,), jnp.float32),
        jax.ShapeDtypeStruct((N,), jnp.int32),
        jax.ShapeDtypeStruct((NW, G), jnp.float32),
        jax.ShapeDtypeStruct((NW, G), jnp.int32),
    ),
    mesh=_mesh,
    compiler_params=pltpu.CompilerParams(needs_layout_passes=False),
    scratch_types=(
        pltpu.VMEM((4, CHUNK), jnp.float32),
        pltpu.VMEM((4, G), jnp.float32),
        pltpu.VMEM((G,), jnp.float32),
        pltpu.VMEM((CHUNK,), jnp.float32),
        pltpu.VMEM((CHUNK,), jnp.int32),
        pltpu.VMEM((G,), jnp.float32),
        pltpu.VMEM((G,), jnp.int32),
        pltpu.SemaphoreType.DMA,
    ),
)(_phase1_body)


def _phase2_body(cm_hbm, ca_hbm, asg_hbm, lab_hbm, aout_hbm, lout_hbm,
                 cm_v, ca_v, lab_v, prev_v, asg_v, lout_v, sem):
    wid, base = _worker_base()
    c1 = pltpu.async_copy(cm_hbm, cm_v, sem)
    c2 = pltpu.async_copy(ca_hbm, ca_v, sem)
    c3 = pltpu.async_copy(lab_hbm, lab_v, sem)
    _chunk_in(asg_hbm, prev_v, base, wid)
    _chunk_in(asg_hbm, asg_v, base, wid)
    c1.wait()
    c2.wait()
    c3.wait()

    iota = lax.iota(jnp.int32, L)

    combined = []
    for j in range(GJ):
        s = pl.ds(j * L, L)
        bcm = cm_v[0, s]
        bca = ca_v[0, s]
        for w in range(1, NW):
            ccm = cm_v[w, s]
            cca = ca_v[w, s]
            m = ccm > bcm
            bcm = jnp.where(m, ccm, bcm)
            bca = jnp.where(m, cca, bca)
        combined.append((bcm, bca))

    locs = []
    inrs = []
    vals = []
    for j in range(GJ):
        bcm, bca = combined[j]
        claim = jnp.where(bcm >= 0.5, _ids_plus1(j), 0)
        loc = bca - base
        inr = jnp.logical_and(loc >= 0, loc < CHUNK)
        locc = jnp.clip(loc, 0, CHUNK - 1)
        prev = plsc.load_gather(prev_v, [locc])
        locs.append(locc)
        inrs.append(inr)
        vals.append(jnp.where(claim > 0, claim, prev))

    for lane in range(L):
        lane_sel = iota == lane
        for j in range(GJ):
            plsc.store_scatter(asg_v, [locs[j]], vals[j],
                               mask=jnp.logical_and(lane_sel, inrs[j]))

    def labels(t, _):
        s = pl.ds(t * L, L)
        av = asg_v[s]
        idx = jnp.clip(av - 1, 0, G - 1)
        lb = plsc.load_gather(lab_v, [idx])
        lout_v[s] = jnp.where(av > 0, lb, -1)
        return 0

    lax.fori_loop(0, CHUNK // L, labels, 0)
    _chunk_out(asg_v, aout_hbm, base, wid)
    _chunk_out(lout_v, lout_hbm, base, wid)


_phase2 = functools.partial(
    pl.kernel,
    out_type=(
        jax.ShapeDtypeStruct((N,), jnp.int32),
        jax.ShapeDtypeStruct((N,), jnp.int32),
    ),
    mesh=_mesh,
    compiler_params=pltpu.CompilerParams(needs_layout_passes=False),
    scratch_types=(
        pltpu.VMEM((NW, G), jnp.float32),
        pltpu.VMEM((NW, G), jnp.int32),
        pltpu.VMEM((G,), jnp.int32),
        pltpu.VMEM((CHUNK,), jnp.int32),
        pltpu.VMEM((CHUNK,), jnp.int32),
        pltpu.VMEM((CHUNK,), jnp.int32),
        pltpu.SemaphoreType.DMA,
    ),
)(_phase2_body)


def kernel(bboxes, gt_bboxes, gt_labels):
    pad = jnp.zeros((NPAD - N, 4), jnp.float32)
    boxes_soa = jnp.concatenate([bboxes, pad], axis=0).T
    gt_soa = gt_bboxes.T.reshape(4, L, GJ).transpose(0, 2, 1).reshape(4, G)
    labels = gt_labels.astype(jnp.int32)
    mo, asg_pre, cm, ca = _phase1(boxes_soa, gt_soa)
    asg, labs = _phase2(cm, ca, asg_pre, labels)
    return asg, mo, labs


# speedup vs baseline: 1.7321x; 1.7321x over previous
import functools

import jax
import jax.numpy as jnp
from jax import lax
from jax.experimental import pallas as pl
from jax.experimental.pallas import tpu as pltpu
from jax.experimental.pallas import tpu_sc as plsc

N = 20000
G = 128
L = 16
GJ = G // L
NC = 2
NS = 16
NW = NC * NS
CHUNK = 640
NPAD = NW * CHUNK

_mesh = plsc.VectorSubcoreMesh(
    core_axis_name="c", subcore_axis_name="s", num_cores=NC, num_subcores=NS)


TAIL = N - (NW - 1) * CHUNK


def _worker_base():
    wid = lax.axis_index("s") * NC + lax.axis_index("c")
    return wid, wid * CHUNK


def _chunk_out(v_ref, hbm_ref, base, wid):
    @pl.when(wid < NW - 1)
    def _():
        pltpu.sync_copy(v_ref, hbm_ref.at[pl.ds(base, CHUNK)])

    @pl.when(wid == NW - 1)
    def _():
        pltpu.sync_copy(v_ref.at[pl.ds(0, TAIL)], hbm_ref.at[pl.ds(base, TAIL)])


def _chunk_in(hbm_ref, v_ref, base, wid):
    @pl.when(wid < NW - 1)
    def _():
        pltpu.sync_copy(hbm_ref.at[pl.ds(base, CHUNK)], v_ref)

    @pl.when(wid == NW - 1)
    def _():
        pltpu.sync_copy(hbm_ref.at[pl.ds(base, TAIL)], v_ref.at[pl.ds(0, TAIL)])


def _ids_plus1(j):
    return lax.iota(jnp.int32, L) * GJ + (j + 1)


def _lane_max(x):
    iota = lax.iota(jnp.int32, L)
    for k in (8, 4, 2, 1):
        perm = jnp.bitwise_xor(iota, k)
        x = jnp.maximum(
            x, jnp.take_along_axis(x, perm, axis=0, mode="promise_in_bounds"))
    return x


def _phase1_body(boxes_hbm, gt_hbm, mo_hbm, asg_hbm, cm_hbm, ca_hbm,
                 bx_v, gt_v, ga_v, rm_v, as_v, cm_v, ca_v, sem):
    wid, base = _worker_base()
    c1 = pltpu.async_copy(boxes_hbm.at[:, pl.ds(base, CHUNK)], bx_v, sem)
    c2 = pltpu.async_copy(gt_hbm, gt_v, sem)
    c1.wait()
    c2.wait()

    iota = lax.iota(jnp.int32, L)
    for j in range(GJ):
        s = pl.ds(j * L, L)
        ga_v[s] = (gt_v[2, s] - gt_v[0, s]) * (gt_v[3, s] - gt_v[1, s])
        ca_v[s] = jnp.zeros((L,), jnp.int32)

    init_cm = tuple(jnp.full((L,), -1.0, jnp.float32) for _ in range(GJ))

    def group(t, cms):
        tb = t * L
        bx1g = bx_v[0, pl.ds(tb, L)]
        by1g = bx_v[1, pl.ds(tb, L)]
        bx2g = bx_v[2, pl.ds(tb, L)]
        by2g = bx_v[3, pl.ds(tb, L)]
        rm_acc = jnp.zeros((L,), jnp.float32)
        as_acc = jnp.zeros((L,), jnp.int32)
        cms = list(cms)
        for half in range(L // 2):
            la, lb = 2 * half, 2 * half + 1
            ax1 = jnp.full((L,), bx1g[la])
            ay1 = jnp.full((L,), by1g[la])
            ax2 = jnp.full((L,), bx2g[la])
            ay2 = jnp.full((L,), by2g[la])
            bx1 = jnp.full((L,), bx1g[lb])
            by1 = jnp.full((L,), by1g[lb])
            bx2 = jnp.full((L,), bx2g[lb])
            by2 = jnp.full((L,), by2g[lb])
            area_a = (ax2 - ax1) * (ay2 - ay1)
            area_b = (bx2 - bx1) * (by2 - by1)
            bidxa = jnp.full((L,), base + tb + la, dtype=jnp.int32)
            bidxb = jnp.full((L,), base + tb + lb, dtype=jnp.int32)
            best_a = best_b = None
            barg_a = barg_b = None
            for j in range(GJ):
                s = pl.ds(j * L, L)
                gx1 = gt_v[0, s]
                gy1 = gt_v[1, s]
                gx2 = gt_v[2, s]
                gy2 = gt_v[3, s]
                ga = ga_v[s]
                ids1 = _ids_plus1(j)

                wa = jnp.maximum(jnp.minimum(ax2, gx2) - jnp.maximum(ax1, gx1),
                                 0.0)
                ha = jnp.maximum(jnp.minimum(ay2, gy2) - jnp.maximum(ay1, gy1),
                                 0.0)
                inter_a = wa * ha
                iou_a = inter_a / (area_a + ga - inter_a)
                ma = iou_a > cms[j]
                cms[j] = jnp.where(ma, iou_a, cms[j])
                if j == 0:
                    best_a, barg_a = iou_a, ids1
                else:
                    ra = iou_a > best_a
                    best_a = jnp.where(ra, iou_a, best_a)
                    barg_a = jnp.where(ra, ids1, barg_a)

                wb = jnp.maximum(jnp.minimum(bx2, gx2) - jnp.maximum(bx1, gx1),
                                 0.0)
                hb = jnp.maximum(jnp.minimum(by2, gy2) - jnp.maximum(by1, gy1),
                                 0.0)
                inter_b = wb * hb
                iou_b = inter_b / (area_b + ga - inter_b)
                mb = iou_b > cms[j]
                cms[j] = jnp.where(mb, iou_b, cms[j])
                ca_v[s] = jnp.where(
                    mb, bidxb, jnp.where(ma, bidxa, ca_v[s]))
                if j == 0:
                    best_b, barg_b = iou_b, ids1
                else:
                    rb = iou_b > best_b
                    best_b = jnp.where(rb, iou_b, best_b)
                    barg_b = jnp.where(rb, ids1, barg_b)

            for lane, best, barg in ((la, best_a, barg_a),
                                     (lb, best_b, barg_b)):
                rmx = _lane_max(best)
                ffs = plsc.all_reduce_ffs(best == rmx)
                argsel = jnp.take_along_axis(barg, ffs, axis=0,
                                             mode="promise_in_bounds")
                asv = jnp.where(rmx >= 0.5, argsel, 0)
                sel = iota == lane
                rm_acc = jnp.where(sel, rmx, rm_acc)
                as_acc = jnp.where(sel, asv, as_acc)
        rm_v[pl.ds(tb, L)] = rm_acc
        as_v[pl.ds(tb, L)] = as_acc
        return tuple(cms)

    cms = lax.fori_loop(0, CHUNK // L, group, init_cm)
    for j in range(GJ):
        cm_v[pl.ds(j * L, L)] = cms[j]
    o3 = pltpu.async_copy(cm_v, cm_hbm.at[wid], sem)
    o4 = pltpu.async_copy(ca_v, ca_hbm.at[wid], sem)
    _chunk_out(rm_v, mo_hbm, base, wid)
    _chunk_out(as_v, asg_hbm, base, wid)
    o3.wait()
    o4.wait()


_phase1 = functools.partial(
    pl.kernel,
    out_type=(
        jax.ShapeDtypeStruct((N,), jnp.float32),
        jax.ShapeDtypeStruct((N,), jnp.int32),
        jax.ShapeDtypeStruct((NW, G), jnp.float32),
        jax.ShapeDtypeStruct((NW, G), jnp.int32),
    ),
    mesh=_mesh,
    compiler_params=pltpu.CompilerParams(needs_layout_passes=False),
    scratch_types=(
        pltpu.VMEM((4, CHUNK), jnp.float32),
        pltpu.VMEM((4, G), jnp.float32),
        pltpu.VMEM((G,), jnp.float32),
        pltpu.VMEM((CHUNK,), jnp.float32),
        pltpu.VMEM((CHUNK,), jnp.int32),
        pltpu.VMEM((G,), jnp.float32),
        pltpu.VMEM((G,), jnp.int32),
        pltpu.SemaphoreType.DMA,
    ),
)(_phase1_body)


def _phase2_body(cm_hbm, ca_hbm, asg_hbm, lab_hbm, aout_hbm, lout_hbm,
                 cm_v, ca_v, lab_v, prev_v, asg_v, lout_v, sem):
    wid, base = _worker_base()
    c1 = pltpu.async_copy(cm_hbm, cm_v, sem)
    c2 = pltpu.async_copy(ca_hbm, ca_v, sem)
    c3 = pltpu.async_copy(lab_hbm, lab_v, sem)
    _chunk_in(asg_hbm, prev_v, base, wid)
    _chunk_in(asg_hbm, asg_v, base, wid)
    c1.wait()
    c2.wait()
    c3.wait()

    iota = lax.iota(jnp.int32, L)

    def comb(w, c):
        new = []
        for j in range(GJ):
            s = pl.ds(j * L, L)
            bcm, bca = c[2 * j], c[2 * j + 1]
            ccm = cm_v[w, s]
            cca = ca_v[w, s]
            m = ccm > bcm
            new.append(jnp.where(m, ccm, bcm))
            new.append(jnp.where(m, cca, bca))
        return tuple(new)

    init = []
    for j in range(GJ):
        s = pl.ds(j * L, L)
        init.append(cm_v[0, s])
        init.append(ca_v[0, s])
    combined = lax.fori_loop(1, NW, comb, tuple(init))

    locs = []
    inrs = []
    vals = []
    for j in range(GJ):
        bcm, bca = combined[2 * j], combined[2 * j + 1]
        claim = jnp.where(bcm >= 0.5, _ids_plus1(j), 0)
        loc = bca - base
        inr = jnp.logical_and(loc >= 0, loc < CHUNK)
        locc = jnp.clip(loc, 0, CHUNK - 1)
        prev = plsc.load_gather(prev_v, [locc])
        locs.append(locc)
        inrs.append(inr)
        vals.append(jnp.where(claim > 0, claim, prev))

    for lane in range(L):
        lane_sel = iota == lane
        for j in range(GJ):
            plsc.store_scatter(asg_v, [locs[j]], vals[j],
                               mask=jnp.logical_and(lane_sel, inrs[j]))

    def labels(t, _):
        s = pl.ds(t * L, L)
        av = asg_v[s]
        idx = jnp.clip(av - 1, 0, G - 1)
        lb = plsc.load_gather(lab_v, [idx])
        lout_v[s] = jnp.where(av > 0, lb, -1)
        return 0

    lax.fori_loop(0, CHUNK // L, labels, 0)
    _chunk_out(asg_v, aout_hbm, base, wid)
    _chunk_out(lout_v, lout_hbm, base, wid)


_phase2 = functools.partial(
    pl.kernel,
    out_type=(
        jax.ShapeDtypeStruct((N,), jnp.int32),
        jax.ShapeDtypeStruct((N,), jnp.int32),
    ),
    mesh=_mesh,
    compiler_params=pltpu.CompilerParams(needs_layout_passes=False),
    scratch_types=(
        pltpu.VMEM((NW, G), jnp.float32),
        pltpu.VMEM((NW, G), jnp.int32),
        pltpu.VMEM((G,), jnp.int32),
        pltpu.VMEM((CHUNK,), jnp.int32),
        pltpu.VMEM((CHUNK,), jnp.int32),
        pltpu.VMEM((CHUNK,), jnp.int32),
        pltpu.SemaphoreType.DMA,
    ),
)(_phase2_body)


def kernel(bboxes, gt_bboxes, gt_labels):
    pad = jnp.zeros((NPAD - N, 4), jnp.float32)
    boxes_soa = jnp.concatenate([bboxes, pad], axis=0).T
    gt_soa = gt_bboxes.T.reshape(4, L, GJ).transpose(0, 2, 1).reshape(4, G)
    labels = gt_labels.astype(jnp.int32)
    mo, asg_pre, cm, ca = _phase1(boxes_soa, gt_soa)
    asg, labs = _phase2(cm, ca, asg_pre, labels)
    return asg, mo, labs
